# Initial kernel scaffold; baseline (speedup 1.0000x reference)
import jax
import jax.numpy as jnp
from jax.experimental import pallas as pl


def kernel(H, D, h_ebd, d_ebd):
    raise NotImplementedError("write your pallas kernel here")


# speedup vs baseline: 1.0069x; 1.0069x over previous
import jax
import jax.numpy as jnp
from jax import lax
from jax.experimental import pallas as pl
from jax.experimental.pallas import tpu as pltpu
from jax.experimental.pallas import tpu_sc as plsc

N_COMPONENTS = 3
N_NODES = 883
N_TIMESTEPS = 12
DIM = N_COMPONENTS * N_NODES * N_TIMESTEPS
BATCH = 1024

NC = 2
NS = 16
LANES = 16
NW = NC * NS
ROWS_PER_W = BATCH // NW
GROUP = 16
CW = 3456
NCH = 9
TAIL_OFF = NCH * CW
TAIL_W = DIM - TAIL_OFF
TAIL_GW = 768
DIM_PAD = TAIL_OFF + TAIL_GW


def _te_body(h_hbm, d_hbm, hidx_hbm, didx_hbm, out_hbm,
             hidx_v, didx_v, hbuf, dbuf, tbuf, sem):
    wid = lax.axis_index("s") * NC + lax.axis_index("c")
    base = wid * ROWS_PER_W

    pltpu.sync_copy(hidx_hbm.at[pl.ds(base, ROWS_PER_W)], hidx_v)
    pltpu.sync_copy(didx_hbm.at[pl.ds(base, ROWS_PER_W)], didx_v)

    def group_step(g, carry):
        rows0 = base + g * GROUP
        hidx = hidx_v.at[pl.ds(g * GROUP, GROUP)]
        didx = didx_v.at[pl.ds(g * GROUP, GROUP)]

        def chunk_step(c, carry2):
            c0 = c * CW
            cp_h = pltpu.async_copy(h_hbm.at[hidx, pl.ds(c0, CW)], hbuf, sem)
            cp_d = pltpu.async_copy(d_hbm.at[didx, pl.ds(c0, CW)], dbuf, sem)
            cp_h.wait()
            cp_d.wait()

            def add_row(r, carry3):
                def add_vec(k, carry4):
                    off = k * LANES
                    hbuf[r, pl.ds(off, LANES)] = (
                        hbuf[r, pl.ds(off, LANES)]
                        + dbuf[r, pl.ds(off, LANES)])
                    return carry4
                lax.fori_loop(0, CW // LANES, add_vec, 0, unroll=8)
                return carry3

            lax.fori_loop(0, GROUP, add_row, 0)
            pltpu.sync_copy(
                hbuf, out_hbm.at[pl.ds(rows0, GROUP), pl.ds(c0, CW)])
            return carry2

        lax.fori_loop(0, NCH, chunk_step, 0)

        cp_h = pltpu.async_copy(
            h_hbm.at[hidx, pl.ds(TAIL_OFF, TAIL_GW)],
            hbuf.at[:, pl.ds(0, TAIL_GW)], sem)
        cp_d = pltpu.async_copy(
            d_hbm.at[didx, pl.ds(TAIL_OFF, TAIL_GW)],
            dbuf.at[:, pl.ds(0, TAIL_GW)], sem)
        cp_h.wait()
        cp_d.wait()

        def tail_row(r, carry2):
            def tail_vec(k, carry3):
                off = k * LANES
                tbuf[r, pl.ds(off, LANES)] = (
                    hbuf[r, pl.ds(off, LANES)]
                    + dbuf[r, pl.ds(off, LANES)])
                return carry3
            lax.fori_loop(0, (TAIL_W + LANES - 1) // LANES, tail_vec, 0,
                          unroll=8)
            return carry2

        lax.fori_loop(0, GROUP, tail_row, 0)
        pltpu.sync_copy(
            tbuf, out_hbm.at[pl.ds(rows0, GROUP), pl.ds(TAIL_OFF, TAIL_W)])
        return carry

    lax.fori_loop(0, ROWS_PER_W // GROUP, group_step, 0)


@jax.jit
def kernel(H, D, h_ebd, d_ebd):
    pad = DIM_PAD - DIM
    hp = jnp.pad(h_ebd, ((0, 0), (0, pad)))
    dp = jnp.pad(d_ebd, ((0, 0), (0, pad)))
    mesh = plsc.VectorSubcoreMesh(core_axis_name="c", subcore_axis_name="s")
    run = pl.kernel(
        _te_body,
        out_type=jax.ShapeDtypeStruct((BATCH, DIM), jnp.float32),
        mesh=mesh,
        scratch_types=[
            pltpu.VMEM((ROWS_PER_W,), jnp.int32),
            pltpu.VMEM((ROWS_PER_W,), jnp.int32),
            pltpu.VMEM((GROUP, CW), jnp.float32),
            pltpu.VMEM((GROUP, CW), jnp.float32),
            pltpu.VMEM((GROUP, TAIL_W), jnp.float32),
            pltpu.SemaphoreType.DMA,
        ],
    )
    out = run(hp, dp, H.astype(jnp.int32), D.astype(jnp.int32))
    return out.reshape(BATCH, N_COMPONENTS, N_NODES, N_TIMESTEPS)


# speedup vs baseline: 1.4294x; 1.4196x over previous
import jax
import jax.numpy as jnp
from jax import lax
from jax.experimental import pallas as pl
from jax.experimental.pallas import tpu as pltpu
from jax.experimental.pallas import tpu_sc as plsc

N_COMPONENTS = 3
N_NODES = 883
N_TIMESTEPS = 12
DIM = N_COMPONENTS * N_NODES * N_TIMESTEPS
BATCH = 1024

NC = 2
NS = 16
LANES = 16
NW = NC * NS
N_SLABS = N_COMPONENTS * N_TIMESTEPS
N_ENTRIES = 24 + 7
N_PAD = 888
B_CHUNKS = 8
BW = BATCH // B_CHUNKS
S_RANGES = 4
SPR = N_SLABS // S_RANGES
NCHK = 112
N_FULL = N_NODES // NCHK
N_TAILW = N_NODES - N_FULL * NCHK


def _te_body(tab_hbm, hidx_hbm, didx_hbm, out_hbm, hc, dc, tbuf, obuf, sem):
    wid = lax.axis_index("s") * NC + lax.axis_index("c")
    bi = wid % B_CHUNKS
    sj = wid // B_CHUNKS
    b0 = bi * BW

    pltpu.sync_copy(hidx_hbm.at[pl.ds(b0, BW)], hc)
    pltpu.sync_copy(didx_hbm.at[pl.ds(b0, BW)], dc)

    hvecs = [hc[pl.ds(k * LANES, LANES)] * N_PAD
             for k in range(BW // LANES)]
    dvecs = [(dc[pl.ds(k * LANES, LANES)] + 24) * N_PAD
             for k in range(BW // LANES)]

    def slab_step(si, carry):
        s = sj * SPR + si
        pltpu.sync_copy(tab_hbm.at[s], tbuf)

        def make_node_step(n0):
            def node_step(nl, carry3):
                nvec = jnp.full((LANES,), n0 + nl, dtype=jnp.int32)
                for k in range(BW // LANES):
                    v = (plsc.load_gather(tbuf, [hvecs[k] + nvec])
                         + plsc.load_gather(tbuf, [dvecs[k] + nvec]))
                    obuf[nl, pl.ds(k * LANES, LANES)] = v
                return carry3
            return node_step

        def chunk_step(ci, carry2):
            n0 = ci * NCHK
            lax.fori_loop(0, NCHK, make_node_step(n0), 0)
            pltpu.sync_copy(
                obuf.at[pl.ds(0, NCHK), :],
                out_hbm.at[s, pl.ds(n0, NCHK), pl.ds(b0, BW)])
            return carry2

        lax.fori_loop(0, N_FULL, chunk_step, 0)
        lax.fori_loop(0, N_TAILW, make_node_step(N_FULL * NCHK), 0)
        pltpu.sync_copy(
            obuf.at[pl.ds(0, N_TAILW), :],
            out_hbm.at[s, pl.ds(N_FULL * NCHK, N_TAILW), pl.ds(b0, BW)])
        return carry

    lax.fori_loop(0, SPR, slab_step, 0)


@jax.jit
def kernel(H, D, h_ebd, d_ebd):
    ht = h_ebd.reshape(24, N_COMPONENTS, N_NODES, N_TIMESTEPS)
    ht = ht.transpose(1, 3, 0, 2)
    dt = d_ebd.reshape(7, N_COMPONENTS, N_NODES, N_TIMESTEPS)
    dt = dt.transpose(1, 3, 0, 2)
    tab = jnp.concatenate([ht, dt], axis=2)
    tab = jnp.pad(tab, ((0, 0), (0, 0), (0, 0), (0, N_PAD - N_NODES)))
    tab = tab.reshape(N_SLABS, N_ENTRIES * N_PAD)

    mesh = plsc.VectorSubcoreMesh(core_axis_name="c", subcore_axis_name="s")
    run = pl.kernel(
        _te_body,
        out_type=jax.ShapeDtypeStruct((N_SLABS, N_NODES, BATCH), jnp.float32),
        mesh=mesh,
        compiler_params=pltpu.CompilerParams(needs_layout_passes=False),
        scratch_types=[
            pltpu.VMEM((BW,), jnp.int32),
            pltpu.VMEM((BW,), jnp.int32),
            pltpu.VMEM((N_ENTRIES * N_PAD,), jnp.float32),
            pltpu.VMEM((NCHK, BW), jnp.float32),
            pltpu.SemaphoreType.DMA,
        ],
    )
    out3 = run(tab, H.astype(jnp.int32), D.astype(jnp.int32))
    out4 = out3.reshape(N_COMPONENTS, N_TIMESTEPS, N_NODES, BATCH)
    return out4.transpose(3, 0, 2, 1)


# speedup vs baseline: 2.7117x; 1.8971x over previous
import jax
import jax.numpy as jnp
from jax import lax
from jax.experimental import pallas as pl
from jax.experimental.pallas import tpu as pltpu
from jax.experimental.pallas import tpu_sc as plsc

N_COMPONENTS = 3
N_NODES = 883
N_TIMESTEPS = 12
DIM = N_COMPONENTS * N_NODES * N_TIMESTEPS
BATCH = 1024

NC = 2
NS = 16
LANES = 16
NW = NC * NS
N_SLABS = N_COMPONENTS * N_TIMESTEPS
N_ENTRIES = 24 + 7
N_PAD = 888
B_CHUNKS = 8
BW = BATCH // B_CHUNKS
S_RANGES = 4
SPR = N_SLABS // S_RANGES
NCHK = 112
N_FULL = N_NODES // NCHK
N_TAILW = N_NODES - N_FULL * NCHK


def _te_body(tab_hbm, hidx_hbm, didx_hbm, out_hbm, hc, dc, tbuf, obuf, sem):
    wid = lax.axis_index("s") * NC + lax.axis_index("c")
    bi = wid % B_CHUNKS
    sj = wid // B_CHUNKS
    b0 = bi * BW

    pltpu.sync_copy(hidx_hbm.at[pl.ds(b0, BW)], hc)
    pltpu.sync_copy(didx_hbm.at[pl.ds(b0, BW)], dc)

    hvecs = [hc[pl.ds(k * LANES, LANES)] * N_PAD
             for k in range(BW // LANES)]
    dvecs = [(dc[pl.ds(k * LANES, LANES)] + 24) * N_PAD
             for k in range(BW // LANES)]

    def slab_step(si, carry):
        s = sj * SPR + si
        pltpu.sync_copy(tab_hbm.at[s], tbuf)

        def run_nodes(n0, nrows):
            @plsc.parallel_loop(0, nrows, unroll=4)
            def node_step(nl):
                nvec = jnp.full((LANES,), n0 + nl, dtype=jnp.int32)
                for k in range(BW // LANES):
                    v = (plsc.load_gather(tbuf, [hvecs[k] + nvec])
                         + plsc.load_gather(tbuf, [dvecs[k] + nvec]))
                    obuf[nl, pl.ds(k * LANES, LANES)] = v

        def chunk_step(ci, carry2):
            n0 = ci * NCHK
            run_nodes(n0, NCHK)
            pltpu.sync_copy(
                obuf.at[pl.ds(0, NCHK), :],
                out_hbm.at[s, pl.ds(n0, NCHK), pl.ds(b0, BW)])
            return carry2

        lax.fori_loop(0, N_FULL, chunk_step, 0)
        run_nodes(N_FULL * NCHK, N_TAILW)
        pltpu.sync_copy(
            obuf.at[pl.ds(0, N_TAILW), :],
            out_hbm.at[s, pl.ds(N_FULL * NCHK, N_TAILW), pl.ds(b0, BW)])
        return carry

    lax.fori_loop(0, SPR, slab_step, 0)


@jax.jit
def kernel(H, D, h_ebd, d_ebd):
    ht = h_ebd.reshape(24, N_COMPONENTS, N_NODES, N_TIMESTEPS)
    ht = ht.transpose(1, 3, 0, 2)
    dt = d_ebd.reshape(7, N_COMPONENTS, N_NODES, N_TIMESTEPS)
    dt = dt.transpose(1, 3, 0, 2)
    tab = jnp.concatenate([ht, dt], axis=2)
    tab = jnp.pad(tab, ((0, 0), (0, 0), (0, 0), (0, N_PAD - N_NODES)))
    tab = tab.reshape(N_SLABS, N_ENTRIES * N_PAD)

    mesh = plsc.VectorSubcoreMesh(core_axis_name="c", subcore_axis_name="s")
    run = pl.kernel(
        _te_body,
        out_type=jax.ShapeDtypeStruct((N_SLABS, N_NODES, BATCH), jnp.float32),
        mesh=mesh,
        compiler_params=pltpu.CompilerParams(needs_layout_passes=False),
        scratch_types=[
            pltpu.VMEM((BW,), jnp.int32),
            pltpu.VMEM((BW,), jnp.int32),
            pltpu.VMEM((N_ENTRIES * N_PAD,), jnp.float32),
            pltpu.VMEM((NCHK, BW), jnp.float32),
            pltpu.SemaphoreType.DMA,
        ],
    )
    out3 = run(tab, H.astype(jnp.int32), D.astype(jnp.int32))
    out4 = out3.reshape(N_COMPONENTS, N_TIMESTEPS, N_NODES, BATCH)
    return out4.transpose(3, 0, 2, 1)
